# baseline (speedup 1.0000x reference)
import functools

import jax
import jax.numpy as jnp
from jax import lax
from jax.experimental import pallas as pl
from jax.experimental.pallas import tpu as pltpu
from jax.experimental.pallas import tpu_sc as plsc

E = 8
N = 4096
D = 1024
P = 4

BA = 4096
BC = 256


def _logits_body(x_ref, w_ref, b_ref, perm_ref, lt_ref, permw_ref):
    lt = lax.dot_general(
        w_ref[...], x_ref[...],
        dimension_numbers=(((1,), (1,)), ((), ())),
        preferred_element_type=jnp.float32,
    )
    lt_ref[...] = lt + b_ref[...]
    permw_ref[...] = jnp.mean(perm_ref[...], axis=0)


def _gate_sc_body(lt_hbm, gates_hbm, lg_v, gates_v):
    info = plsc.get_sparse_core_info()
    nc, ns, nl = info.num_cores, info.num_subcores, info.num_lanes
    tpw = N // (nc * ns)
    wid = lax.axis_index("s") * nc + lax.axis_index("c")
    base = wid * tpw
    pltpu.sync_copy(lt_hbm.at[:, pl.ds(base, tpw)], lg_v)
    lane = lax.iota(jnp.int32, nl)
    for g in range(tpw // nl):
        l = [lg_v[e, pl.ds(g * nl, nl)] for e in range(E)]
        best = l[0]
        i1 = jnp.zeros((nl,), jnp.int32)
        for e in range(1, E):
            c = l[e] > best
            best = jnp.where(c, l[e], best)
            i1 = jnp.where(c, e, i1)
        best2 = jnp.full((nl,), -jnp.inf, jnp.float32)
        i2 = jnp.zeros((nl,), jnp.int32)
        for e in range(E):
            c = (i1 != e) & (l[e] > best2)
            best2 = jnp.where(c, l[e], best2)
            i2 = jnp.where(c, e, i2)
        p = []
        for e in range(E):
            sel = ((i1 == e) | (i2 == e)) & (l[e] != 0.0)
            p.append(jnp.where(sel, jnp.exp(l[e] - best), 0.0))
        denom = p[0]
        for e in range(1, E):
            denom = denom + p[e]
        inv = 1.0 / denom
        for e in range(E):
            gates_v[e, pl.ds(g * nl, nl)] = p[e] * inv
    pltpu.sync_copy(gates_v, gates_hbm.at[:, pl.ds(base, tpw)])


def _combine_body(gates_ref, h_ref, permw_ref, y_ref, soft_ref, hard_ref):
    pid = pl.program_id(0)
    nblk = pl.num_programs(0)
    g = jnp.transpose(gates_ref[...])
    gp = lax.dot_general(
        g, permw_ref[...],
        dimension_numbers=(((1,), (0,)), ((), ())),
        preferred_element_type=jnp.float32,
    )
    gpn = gp / jnp.sum(gp, axis=1, keepdims=True)
    acc = gpn[:, 0:1] * h_ref[0]
    for e in range(1, E):
        acc = acc + gpn[:, e:e + 1] * h_ref[e]
    y_ref[...] = acc
    psoft = jnp.sum(gpn, axis=0, keepdims=True)
    phard = jnp.sum(jnp.where(gpn < 1e-5, 0.0, 1.0), axis=0, keepdims=True)

    @pl.when(pid == 0)
    def _init():
        soft_ref[...] = psoft
        hard_ref[...] = phard

    @pl.when(pid != 0)
    def _acc():
        soft_ref[...] += psoft
        hard_ref[...] += phard

    @pl.when(pid == nblk - 1)
    def _fin():
        soft_ref[...] = soft_ref[...] * (1.0 / N)
        hard_ref[...] = hard_ref[...] * (1.0 / N)


def kernel(h, x, permutation_weights, W, b):
    b2 = b.reshape(E, 1)

    logits_t, permw = pl.pallas_call(
        _logits_body,
        grid=(N // BA,),
        in_specs=[
            pl.BlockSpec((BA, D), lambda i: (i, 0)),
            pl.BlockSpec((E, D), lambda i: (0, 0)),
            pl.BlockSpec((E, 1), lambda i: (0, 0)),
            pl.BlockSpec((P, E, E), lambda i: (0, 0, 0)),
        ],
        out_specs=[
            pl.BlockSpec((E, BA), lambda i: (0, i)),
            pl.BlockSpec((E, E), lambda i: (0, 0)),
        ],
        out_shape=[
            jax.ShapeDtypeStruct((E, N), jnp.float32),
            jax.ShapeDtypeStruct((E, E), jnp.float32),
        ],
    )(x, W, b2, permutation_weights)

    info = plsc.get_sparse_core_info()
    tpw = N // (info.num_cores * info.num_subcores)
    gates = pl.kernel(
        _gate_sc_body,
        out_type=jax.ShapeDtypeStruct((E, N), jnp.float32),
        mesh=plsc.VectorSubcoreMesh(core_axis_name="c", subcore_axis_name="s"),
        scratch_types=[
            pltpu.VMEM((E, tpw), jnp.float32),
            pltpu.VMEM((E, tpw), jnp.float32),
        ],
    )(logits_t)

    y, soft, hard = pl.pallas_call(
        _combine_body,
        grid=(N // BC,),
        in_specs=[
            pl.BlockSpec((E, BC), lambda i: (0, i)),
            pl.BlockSpec((E, BC, D), lambda i: (0, i, 0)),
            pl.BlockSpec((E, E), lambda i: (0, 0)),
        ],
        out_specs=[
            pl.BlockSpec((BC, D), lambda i: (i, 0)),
            pl.BlockSpec((1, E), lambda i: (0, 0)),
            pl.BlockSpec((1, E), lambda i: (0, 0)),
        ],
        out_shape=[
            jax.ShapeDtypeStruct((N, D), jnp.float32),
            jax.ShapeDtypeStruct((1, E), jnp.float32),
            jax.ShapeDtypeStruct((1, E), jnp.float32),
        ],
    )(gates, h, permw)

    return (y, soft.reshape(E, 1), hard.reshape(E, 1))


# speedup vs baseline: 1.0073x; 1.0073x over previous
import functools

import jax
import jax.numpy as jnp
from jax import lax
from jax.experimental import pallas as pl
from jax.experimental.pallas import tpu as pltpu
from jax.experimental.pallas import tpu_sc as plsc

E = 8
N = 4096
D = 1024
P = 4

BA = 2048
BC = 512


def _logits_body(x_ref, w_ref, b_ref, perm_ref, lt_ref, permw_ref):
    lt = lax.dot_general(
        w_ref[...], x_ref[...],
        dimension_numbers=(((1,), (1,)), ((), ())),
        preferred_element_type=jnp.float32,
    )
    lt_ref[...] = lt + b_ref[...]
    permw_ref[...] = jnp.mean(perm_ref[...], axis=0)


def _gate_sc_body(lt_hbm, gates_hbm, lg_v, gates_v):
    info = plsc.get_sparse_core_info()
    nc, ns, nl = info.num_cores, info.num_subcores, info.num_lanes
    tpw = N // (nc * ns)
    wid = lax.axis_index("s") * nc + lax.axis_index("c")
    base = wid * tpw
    pltpu.sync_copy(lt_hbm.at[:, pl.ds(base, tpw)], lg_v)
    lane = lax.iota(jnp.int32, nl)
    for g in range(tpw // nl):
        l = [lg_v[e, pl.ds(g * nl, nl)] for e in range(E)]
        best = l[0]
        i1 = jnp.zeros((nl,), jnp.int32)
        for e in range(1, E):
            c = l[e] > best
            best = jnp.where(c, l[e], best)
            i1 = jnp.where(c, e, i1)
        best2 = jnp.full((nl,), -jnp.inf, jnp.float32)
        i2 = jnp.zeros((nl,), jnp.int32)
        for e in range(E):
            c = (i1 != e) & (l[e] > best2)
            best2 = jnp.where(c, l[e], best2)
            i2 = jnp.where(c, e, i2)
        p = []
        for e in range(E):
            sel = ((i1 == e) | (i2 == e)) & (l[e] != 0.0)
            p.append(jnp.where(sel, jnp.exp(l[e] - best), 0.0))
        denom = p[0]
        for e in range(1, E):
            denom = denom + p[e]
        inv = 1.0 / denom
        for e in range(E):
            gates_v[e, pl.ds(g * nl, nl)] = p[e] * inv
    pltpu.sync_copy(gates_v, gates_hbm.at[:, pl.ds(base, tpw)])


def _combine_body(gates_ref, h_ref, permw_ref, y_ref, soft_ref, hard_ref):
    pid = pl.program_id(0)
    nblk = pl.num_programs(0)
    g = jnp.transpose(gates_ref[...])
    gp = lax.dot_general(
        g, permw_ref[...],
        dimension_numbers=(((1,), (0,)), ((), ())),
        preferred_element_type=jnp.float32,
    )
    gpn = gp / jnp.sum(gp, axis=1, keepdims=True)
    acc = gpn[:, 0:1] * h_ref[0]
    for e in range(1, E):
        acc = acc + gpn[:, e:e + 1] * h_ref[e]
    y_ref[...] = acc
    psoft = jnp.sum(gpn, axis=0, keepdims=True)
    phard = jnp.sum(jnp.where(gpn < 1e-5, 0.0, 1.0), axis=0, keepdims=True)

    @pl.when(pid == 0)
    def _init():
        soft_ref[...] = psoft
        hard_ref[...] = phard

    @pl.when(pid != 0)
    def _acc():
        soft_ref[...] += psoft
        hard_ref[...] += phard

    @pl.when(pid == nblk - 1)
    def _fin():
        soft_ref[...] = soft_ref[...] * (1.0 / N)
        hard_ref[...] = hard_ref[...] * (1.0 / N)


def kernel(h, x, permutation_weights, W, b):
    b2 = b.reshape(E, 1)

    logits_t, permw = pl.pallas_call(
        _logits_body,
        grid=(N // BA,),
        in_specs=[
            pl.BlockSpec((BA, D), lambda i: (i, 0)),
            pl.BlockSpec((E, D), lambda i: (0, 0)),
            pl.BlockSpec((E, 1), lambda i: (0, 0)),
            pl.BlockSpec((P, E, E), lambda i: (0, 0, 0)),
        ],
        out_specs=[
            pl.BlockSpec((E, BA), lambda i: (0, i)),
            pl.BlockSpec((E, E), lambda i: (0, 0)),
        ],
        out_shape=[
            jax.ShapeDtypeStruct((E, N), jnp.float32),
            jax.ShapeDtypeStruct((E, E), jnp.float32),
        ],
    )(x, W, b2, permutation_weights)

    info = plsc.get_sparse_core_info()
    tpw = N // (info.num_cores * info.num_subcores)
    gates = pl.kernel(
        _gate_sc_body,
        out_type=jax.ShapeDtypeStruct((E, N), jnp.float32),
        mesh=plsc.VectorSubcoreMesh(core_axis_name="c", subcore_axis_name="s"),
        scratch_types=[
            pltpu.VMEM((E, tpw), jnp.float32),
            pltpu.VMEM((E, tpw), jnp.float32),
        ],
    )(logits_t)

    y, soft, hard = pl.pallas_call(
        _combine_body,
        grid=(N // BC,),
        in_specs=[
            pl.BlockSpec((E, BC), lambda i: (0, i)),
            pl.BlockSpec((E, BC, D), lambda i: (0, i, 0)),
            pl.BlockSpec((E, E), lambda i: (0, 0)),
        ],
        out_specs=[
            pl.BlockSpec((BC, D), lambda i: (i, 0)),
            pl.BlockSpec((1, E), lambda i: (0, 0)),
            pl.BlockSpec((1, E), lambda i: (0, 0)),
        ],
        out_shape=[
            jax.ShapeDtypeStruct((N, D), jnp.float32),
            jax.ShapeDtypeStruct((1, E), jnp.float32),
            jax.ShapeDtypeStruct((1, E), jnp.float32),
        ],
    )(gates, h, permw)

    return (y, soft.reshape(E, 1), hard.reshape(E, 1))


# speedup vs baseline: 1.0143x; 1.0070x over previous
import functools

import jax
import jax.numpy as jnp
from jax import lax
from jax.experimental import pallas as pl
from jax.experimental.pallas import tpu as pltpu
from jax.experimental.pallas import tpu_sc as plsc

E = 8
N = 4096
D = 1024
P = 4

BA = 2048
BC = 256


def _logits_body(x_ref, w_ref, b_ref, perm_ref, lt_ref, permw_ref):
    lt = lax.dot_general(
        w_ref[...], x_ref[...],
        dimension_numbers=(((1,), (1,)), ((), ())),
        preferred_element_type=jnp.float32,
    )
    lt_ref[...] = lt + b_ref[...]
    permw_ref[...] = jnp.mean(perm_ref[...], axis=0)


def _gate_sc_body(lt_hbm, gates_hbm, lg_v, gates_v):
    info = plsc.get_sparse_core_info()
    nc, ns, nl = info.num_cores, info.num_subcores, info.num_lanes
    tpw = N // (nc * ns)
    wid = lax.axis_index("s") * nc + lax.axis_index("c")
    base = wid * tpw
    pltpu.sync_copy(lt_hbm.at[:, pl.ds(base, tpw)], lg_v)
    lane = lax.iota(jnp.int32, nl)
    for g in range(tpw // nl):
        l = [lg_v[e, pl.ds(g * nl, nl)] for e in range(E)]
        best = l[0]
        i1 = jnp.zeros((nl,), jnp.int32)
        for e in range(1, E):
            c = l[e] > best
            best = jnp.where(c, l[e], best)
            i1 = jnp.where(c, e, i1)
        best2 = jnp.full((nl,), -jnp.inf, jnp.float32)
        i2 = jnp.zeros((nl,), jnp.int32)
        for e in range(E):
            c = (i1 != e) & (l[e] > best2)
            best2 = jnp.where(c, l[e], best2)
            i2 = jnp.where(c, e, i2)
        p = []
        for e in range(E):
            sel = ((i1 == e) | (i2 == e)) & (l[e] != 0.0)
            p.append(jnp.where(sel, jnp.exp(l[e] - best), 0.0))
        denom = p[0]
        for e in range(1, E):
            denom = denom + p[e]
        inv = 1.0 / denom
        for e in range(E):
            gates_v[e, pl.ds(g * nl, nl)] = p[e] * inv
    pltpu.sync_copy(gates_v, gates_hbm.at[:, pl.ds(base, tpw)])


def _combine_body(gates_ref, h_ref, permw_ref, y_ref, soft_ref, hard_ref):
    pid = pl.program_id(0)
    nblk = pl.num_programs(0)
    g = jnp.transpose(gates_ref[...])
    gp = lax.dot_general(
        g, permw_ref[...],
        dimension_numbers=(((1,), (0,)), ((), ())),
        preferred_element_type=jnp.float32,
    )
    gpn = gp / jnp.sum(gp, axis=1, keepdims=True)
    acc = gpn[:, 0:1] * h_ref[0]
    for e in range(1, E):
        acc = acc + gpn[:, e:e + 1] * h_ref[e]
    y_ref[...] = acc
    psoft = jnp.sum(gpn, axis=0, keepdims=True)
    phard = jnp.sum(jnp.where(gpn < 1e-5, 0.0, 1.0), axis=0, keepdims=True)

    @pl.when(pid == 0)
    def _init():
        soft_ref[...] = psoft
        hard_ref[...] = phard

    @pl.when(pid != 0)
    def _acc():
        soft_ref[...] += psoft
        hard_ref[...] += phard

    @pl.when(pid == nblk - 1)
    def _fin():
        soft_ref[...] = soft_ref[...] * (1.0 / N)
        hard_ref[...] = hard_ref[...] * (1.0 / N)


def kernel(h, x, permutation_weights, W, b):
    b2 = b.reshape(E, 1)

    logits_t, permw = pl.pallas_call(
        _logits_body,
        grid=(N // BA,),
        in_specs=[
            pl.BlockSpec((BA, D), lambda i: (i, 0)),
            pl.BlockSpec((E, D), lambda i: (0, 0)),
            pl.BlockSpec((E, 1), lambda i: (0, 0)),
            pl.BlockSpec((P, E, E), lambda i: (0, 0, 0)),
        ],
        out_specs=[
            pl.BlockSpec((E, BA), lambda i: (0, i)),
            pl.BlockSpec((E, E), lambda i: (0, 0)),
        ],
        out_shape=[
            jax.ShapeDtypeStruct((E, N), jnp.float32),
            jax.ShapeDtypeStruct((E, E), jnp.float32),
        ],
    )(x, W, b2, permutation_weights)

    info = plsc.get_sparse_core_info()
    tpw = N // (info.num_cores * info.num_subcores)
    gates = pl.kernel(
        _gate_sc_body,
        out_type=jax.ShapeDtypeStruct((E, N), jnp.float32),
        mesh=plsc.VectorSubcoreMesh(core_axis_name="c", subcore_axis_name="s"),
        scratch_types=[
            pltpu.VMEM((E, tpw), jnp.float32),
            pltpu.VMEM((E, tpw), jnp.float32),
        ],
    )(logits_t)

    y, soft, hard = pl.pallas_call(
        _combine_body,
        grid=(N // BC,),
        in_specs=[
            pl.BlockSpec((E, BC), lambda i: (0, i)),
            pl.BlockSpec((E, BC, D), lambda i: (0, i, 0)),
            pl.BlockSpec((E, E), lambda i: (0, 0)),
        ],
        out_specs=[
            pl.BlockSpec((BC, D), lambda i: (i, 0)),
            pl.BlockSpec((1, E), lambda i: (0, 0)),
            pl.BlockSpec((1, E), lambda i: (0, 0)),
        ],
        out_shape=[
            jax.ShapeDtypeStruct((N, D), jnp.float32),
            jax.ShapeDtypeStruct((1, E), jnp.float32),
            jax.ShapeDtypeStruct((1, E), jnp.float32),
        ],
    )(gates, h, permw)

    return (y, soft.reshape(E, 1), hard.reshape(E, 1))
